# Initial kernel scaffold; baseline (speedup 1.0000x reference)
import jax
import jax.numpy as jnp
from jax.experimental import pallas as pl


def kernel(input, adj, W, b):
    raise NotImplementedError("write your pallas kernel here")


# speedup vs baseline: 1.2329x; 1.2329x over previous
import functools

import jax
import jax.numpy as jnp
from jax.experimental import pallas as pl
from jax.experimental.pallas import tpu as pltpu

_N = 10000
_NH = 128

_IB = 2000
_JB = 512
_NI = _N // _IB
_NJ = -(-_N // _JB)


def _support_body(x_ref, w_ref, b_ref, o_ref):
    acc = jax.lax.dot_general(
        x_ref[...], w_ref[...], (((1,), (1,)), ((), ())),
        preferred_element_type=jnp.float32)
    o_ref[...] = jnp.maximum(acc + b_ref[...], 0.0).astype(jnp.bfloat16)


def _agg_body(adj_ref, sup_ref, o_ref, acc_ref, deg_ref, *, n_i):
    i = pl.program_id(1)

    @pl.when(i == 0)
    def _zero():
        acc_ref[...] = jnp.zeros_like(acc_ref)
        deg_ref[...] = jnp.zeros_like(deg_ref)

    mask = (adj_ref[...] > 0.0).astype(jnp.bfloat16)
    sup = sup_ref[pl.ds(i * _IB, _IB), :]
    acc_ref[...] += jax.lax.dot_general(
        mask, sup, (((0,), (0,)), ((), ())),
        preferred_element_type=jnp.float32)
    ones = jnp.ones((_IB, _NH), dtype=jnp.bfloat16)
    deg_ref[...] += jax.lax.dot_general(
        mask, ones, (((0,), (0,)), ((), ())),
        preferred_element_type=jnp.float32)

    @pl.when(i == n_i - 1)
    def _emit():
        o_ref[...] = acc_ref[...] / deg_ref[...]


def kernel(input, adj, W, b):
    support = pl.pallas_call(
        _support_body,
        out_shape=jax.ShapeDtypeStruct((_N, _NH), jnp.bfloat16),
    )(input, W, b.reshape(1, _NH))

    return pl.pallas_call(
        functools.partial(_agg_body, n_i=_NI),
        grid=(_NJ, _NI),
        in_specs=[
            pl.BlockSpec((_IB, _JB), lambda j, i: (i, j)),
            pl.BlockSpec((_N, _NH), lambda j, i: (0, 0)),
        ],
        out_specs=pl.BlockSpec((_JB, _NH), lambda j, i: (j, 0)),
        out_shape=jax.ShapeDtypeStruct((_N, _NH), jnp.float32),
        scratch_shapes=[
            pltpu.VMEM((_JB, _NH), jnp.float32),
            pltpu.VMEM((_JB, _NH), jnp.float32),
        ],
        compiler_params=pltpu.CompilerParams(
            dimension_semantics=("parallel", "arbitrary"),
        ),
    )(adj, support)


# speedup vs baseline: 1.4357x; 1.1645x over previous
import functools

import jax
import jax.numpy as jnp
from jax.experimental import pallas as pl
from jax.experimental.pallas import tpu as pltpu

_N = 10000
_NH = 128

_IB = 2000
_JB = 512
_NI = _N // _IB
_NJ = -(-_N // _JB)


def _support_body(x_ref, w_ref, b_ref, o_ref):
    acc = jax.lax.dot_general(
        x_ref[...], w_ref[...], (((1,), (1,)), ((), ())),
        preferred_element_type=jnp.float32)
    o_ref[...] = jnp.maximum(acc + b_ref[...], 0.0).astype(jnp.bfloat16)


def _agg_body(adj_ref, sup_ref, o_ref, acc_ref, deg_ref, *, n_i):
    i = pl.program_id(1)

    @pl.when(i == 0)
    def _zero():
        acc_ref[...] = jnp.zeros_like(acc_ref)
        deg_ref[...] = jnp.zeros_like(deg_ref)

    sel = jnp.where(adj_ref[...] > 0.0, 1.0, 0.0)
    deg_ref[...] += jnp.sum(sel, axis=0, keepdims=True)
    mask = sel.astype(jnp.bfloat16)
    sup = sup_ref[pl.ds(i * _IB, _IB), :]
    acc_ref[...] += jax.lax.dot_general(
        sup, mask, (((0,), (0,)), ((), ())),
        preferred_element_type=jnp.float32)

    @pl.when(i == n_i - 1)
    def _emit():
        o_ref[...] = jnp.transpose(acc_ref[...] / deg_ref[...])


def kernel(input, adj, W, b):
    support = pl.pallas_call(
        _support_body,
        out_shape=jax.ShapeDtypeStruct((_N, _NH), jnp.bfloat16),
    )(input, W, b.reshape(1, _NH))

    return pl.pallas_call(
        functools.partial(_agg_body, n_i=_NI),
        grid=(_NJ, _NI),
        in_specs=[
            pl.BlockSpec((_IB, _JB), lambda j, i: (i, j)),
            pl.BlockSpec((_N, _NH), lambda j, i: (0, 0)),
        ],
        out_specs=pl.BlockSpec((_JB, _NH), lambda j, i: (j, 0)),
        out_shape=jax.ShapeDtypeStruct((_N, _NH), jnp.float32),
        scratch_shapes=[
            pltpu.VMEM((_NH, _JB), jnp.float32),
            pltpu.VMEM((1, _JB), jnp.float32),
        ],
        compiler_params=pltpu.CompilerParams(
            dimension_semantics=("parallel", "arbitrary"),
        ),
    )(adj, support)


# speedup vs baseline: 1.6810x; 1.1709x over previous
import functools

import jax
import jax.numpy as jnp
from jax.experimental import pallas as pl
from jax.experimental.pallas import tpu as pltpu

_N = 10000
_NH = 128

_IB = 200
_NI = _N // _IB


def _support_body(x_ref, w_ref, b_ref, o_ref):
    acc = jax.lax.dot_general(
        x_ref[...], w_ref[...], (((1,), (1,)), ((), ())),
        preferred_element_type=jnp.float32)
    o_ref[...] = jnp.maximum(acc + b_ref[...], 0.0).astype(jnp.bfloat16)


def _agg_body(adj_ref, sup_ref, o_ref, acc_ref, deg_ref, *, n_i):
    i = pl.program_id(0)

    @pl.when(i == 0)
    def _zero():
        acc_ref[...] = jnp.zeros_like(acc_ref)
        deg_ref[...] = jnp.zeros_like(deg_ref)

    sel = jnp.where(adj_ref[...] > 0.0, 1.0, 0.0)
    deg_ref[...] += jnp.sum(sel, axis=0, keepdims=True)
    mask = sel.astype(jnp.bfloat16)
    sup = sup_ref[pl.ds(i * _IB, _IB), :]
    acc_ref[...] += jax.lax.dot_general(
        sup, mask, (((0,), (0,)), ((), ())),
        preferred_element_type=jnp.float32)

    @pl.when(i == n_i - 1)
    def _emit():
        o_ref[...] = jnp.transpose(acc_ref[...] / deg_ref[...])


def kernel(input, adj, W, b):
    support = pl.pallas_call(
        _support_body,
        out_shape=jax.ShapeDtypeStruct((_N, _NH), jnp.bfloat16),
    )(input, W, b.reshape(1, _NH))

    return pl.pallas_call(
        functools.partial(_agg_body, n_i=_NI),
        grid=(_NI,),
        in_specs=[
            pl.BlockSpec((_IB, _N), lambda i: (i, 0)),
            pl.BlockSpec((_N, _NH), lambda i: (0, 0)),
        ],
        out_specs=pl.BlockSpec((_N, _NH), lambda i: (0, 0)),
        out_shape=jax.ShapeDtypeStruct((_N, _NH), jnp.float32),
        scratch_shapes=[
            pltpu.VMEM((_NH, _N), jnp.float32),
            pltpu.VMEM((1, _N), jnp.float32),
        ],
        compiler_params=pltpu.CompilerParams(
            dimension_semantics=("arbitrary",),
        ),
    )(adj, support)


# speedup vs baseline: 1.7934x; 1.0669x over previous
import functools

import jax
import jax.numpy as jnp
from jax.experimental import pallas as pl
from jax.experimental.pallas import tpu as pltpu

_N = 10000
_NH = 128

_IB = 400
_NI = _N // _IB


def _agg_body(adj_ref, x_ref, w_ref, b_ref, o_ref, acc_ref, deg_ref, *, n_i):
    i = pl.program_id(0)

    @pl.when(i == 0)
    def _zero():
        acc_ref[...] = jnp.zeros_like(acc_ref)
        deg_ref[...] = jnp.zeros_like(deg_ref)

    sup = jnp.maximum(
        jax.lax.dot_general(
            x_ref[...], w_ref[...], (((1,), (1,)), ((), ())),
            preferred_element_type=jnp.float32) + b_ref[...],
        0.0).astype(jnp.bfloat16)

    sel = jnp.where(adj_ref[...] > 0.0, 1.0, 0.0)
    deg_ref[...] += jnp.sum(sel, axis=0, keepdims=True)
    mask = sel.astype(jnp.bfloat16)
    acc_ref[...] += jax.lax.dot_general(
        sup, mask, (((0,), (0,)), ((), ())),
        preferred_element_type=jnp.float32)

    @pl.when(i == n_i - 1)
    def _emit():
        o_ref[...] = jnp.transpose(acc_ref[...] / deg_ref[...])


def kernel(input, adj, W, b):
    return pl.pallas_call(
        functools.partial(_agg_body, n_i=_NI),
        grid=(_NI,),
        in_specs=[
            pl.BlockSpec((_IB, _N), lambda i: (i, 0)),
            pl.BlockSpec((_IB, _NH), lambda i: (i, 0)),
            pl.BlockSpec((_NH, _NH), lambda i: (0, 0)),
            pl.BlockSpec((1, _NH), lambda i: (0, 0)),
        ],
        out_specs=pl.BlockSpec((_N, _NH), lambda i: (0, 0)),
        out_shape=jax.ShapeDtypeStruct((_N, _NH), jnp.float32),
        scratch_shapes=[
            pltpu.VMEM((_NH, _N), jnp.float32),
            pltpu.VMEM((1, _N), jnp.float32),
        ],
        compiler_params=pltpu.CompilerParams(
            dimension_semantics=("arbitrary",),
        ),
    )(adj, input, W, b.reshape(1, _NH))


# speedup vs baseline: 1.8468x; 1.0297x over previous
import functools

import jax
import jax.numpy as jnp
from jax.experimental import pallas as pl
from jax.experimental.pallas import tpu as pltpu

_N = 10000
_NH = 128

_IB = 200
_SPS = 2
_RPS = _IB * _SPS
_NI = _N // _RPS


def _agg_body(adj0_ref, adj1_ref, x_ref, w_ref, b_ref, o_ref,
              acc_ref, deg_ref, *, n_i):
    i = pl.program_id(0)

    @pl.when(i == 0)
    def _zero():
        acc_ref[...] = jnp.zeros_like(acc_ref)
        deg_ref[...] = jnp.zeros_like(deg_ref)

    sup = jnp.maximum(
        jax.lax.dot_general(
            x_ref[...], w_ref[...], (((1,), (1,)), ((), ())),
            preferred_element_type=jnp.float32) + b_ref[...],
        0.0).astype(jnp.bfloat16)

    deg = deg_ref[...]
    acc = acc_ref[...]
    for s, stripe_ref in enumerate((adj0_ref, adj1_ref)):
        sel = jnp.where(stripe_ref[...] > 0.0, 1.0, 0.0)
        deg += jnp.sum(sel, axis=0, keepdims=True)
        mask = sel.astype(jnp.bfloat16)
        acc += jax.lax.dot_general(
            sup[s * _IB:(s + 1) * _IB, :], mask, (((0,), (0,)), ((), ())),
            preferred_element_type=jnp.float32)
    deg_ref[...] = deg
    acc_ref[...] = acc

    @pl.when(i == n_i - 1)
    def _emit():
        o_ref[...] = jnp.transpose(acc_ref[...] / deg_ref[...])


def kernel(input, adj, W, b):
    return pl.pallas_call(
        functools.partial(_agg_body, n_i=_NI),
        grid=(_NI,),
        in_specs=[
            pl.BlockSpec((_IB, _N), lambda i: (2 * i, 0)),
            pl.BlockSpec((_IB, _N), lambda i: (2 * i + 1, 0)),
            pl.BlockSpec((_RPS, _NH), lambda i: (i, 0)),
            pl.BlockSpec((_NH, _NH), lambda i: (0, 0)),
            pl.BlockSpec((1, _NH), lambda i: (0, 0)),
        ],
        out_specs=pl.BlockSpec((_N, _NH), lambda i: (0, 0)),
        out_shape=jax.ShapeDtypeStruct((_N, _NH), jnp.float32),
        scratch_shapes=[
            pltpu.VMEM((_NH, _N), jnp.float32),
            pltpu.VMEM((1, _N), jnp.float32),
        ],
        compiler_params=pltpu.CompilerParams(
            dimension_semantics=("arbitrary",),
        ),
    )(adj, adj, input, W, b.reshape(1, _NH))
